# Initial kernel scaffold; baseline (speedup 1.0000x reference)
import jax
import jax.numpy as jnp
from jax.experimental import pallas as pl


def kernel(x, edge_index, W1_l, b1, W1_r, W2_l, b2, W2_r):
    raise NotImplementedError("write your pallas kernel here")


# speedup vs baseline: 6.2999x; 6.2999x over previous
import functools

import jax
import jax.numpy as jnp
from jax import lax
from jax.experimental import pallas as pl
from jax.experimental.pallas import tpu as pltpu
from jax.experimental.pallas import tpu_sc as plsc

NC = 2
NS = 16
CHUNK = 128


def _make_aggregate(n, e, d, with_deg):
    nw = NC * NS
    n_chunks = e // CHUNK
    rpt = n // NS
    assert n % NS == 0 and e % CHUNK == 0

    out_type = [jax.ShapeDtypeStruct((2 * n, d), jnp.float32)]
    scratch = [
        pltpu.VMEM((1, CHUNK), jnp.int32),
        pltpu.VMEM((1, CHUNK), jnp.int32),
        pltpu.VMEM((CHUNK, d), jnp.float32),
        pltpu.VMEM_SHARED((n, d), jnp.float32),
        pltpu.SemaphoreType.DMA,
    ]
    if with_deg:
        out_type.append(jax.ShapeDtypeStruct((2 * n, 16), jnp.float32))
        scratch += [
            pltpu.VMEM((CHUNK, 16), jnp.float32),
            pltpu.VMEM_SHARED((n, 16), jnp.float32),
        ]

    mesh = plsc.VectorSubcoreMesh(core_axis_name="c", subcore_axis_name="s")

    def body(feat_hbm, src_hbm, dst_hbm, zacc_hbm, *rest):
        if with_deg:
            (zdeg_hbm, ones_hbm, out_hbm, deg_out,
             src_v, dst_v, rows_v, acc_sh, sem, ones_v, deg_sh) = rest
        else:
            (out_hbm, src_v, dst_v, rows_v, acc_sh, sem) = rest
        c = lax.axis_index("c")
        s = lax.axis_index("s")
        w = s * NC + c

        pltpu.sync_copy(zacc_hbm, acc_sh.at[pl.ds(s * rpt, rpt)])
        if with_deg:
            pltpu.sync_copy(zdeg_hbm, deg_sh.at[pl.ds(s * rpt, rpt)])
            pltpu.sync_copy(ones_hbm, ones_v)
        plsc.subcore_barrier()

        base = n_chunks // nw
        rem = n_chunks % nw
        niter = base + jnp.where(w < rem, 1, 0)

        def step(i, carry):
            chunk = w + i * nw
            pltpu.sync_copy(src_hbm.at[pl.ds(chunk, 1)], src_v)
            pltpu.sync_copy(dst_hbm.at[pl.ds(chunk, 1)], dst_v)
            pltpu.async_copy(feat_hbm.at[src_v.at[0]], rows_v, sem).wait()
            pltpu.sync_copy(rows_v, acc_sh.at[dst_v.at[0]], add=True)
            if with_deg:
                pltpu.sync_copy(ones_v, deg_sh.at[dst_v.at[0]], add=True)
            return carry

        lax.fori_loop(0, niter, step, 0)
        plsc.subcore_barrier()

        pltpu.sync_copy(acc_sh.at[pl.ds(s * rpt, rpt)],
                        out_hbm.at[pl.ds(c * n + s * rpt, rpt)])
        if with_deg:
            pltpu.sync_copy(deg_sh.at[pl.ds(s * rpt, rpt)],
                            deg_out.at[pl.ds(c * n + s * rpt, rpt)])

    return pl.kernel(
        body, out_type=out_type, mesh=mesh, scratch_types=scratch,
        compiler_params=pltpu.CompilerParams(use_tc_tiling_on_sc=False))


def _make_dense(n, d, act):
    blk = 1000
    assert n % blk == 0
    grid = (n // blk,)

    def body(p0_r, p1_r, d0_r, d1_r, x_r, wl_r, b_r, wr_r, o_r):
        deg = d0_r[:, :1] + d1_r[:, :1]
        agg = (p0_r[...] + p1_r[...]) / jnp.maximum(deg, 1.0)
        h = (jnp.dot(agg, wl_r[...], preferred_element_type=jnp.float32,
                     precision=lax.Precision.HIGHEST)
             + b_r[...]
             + jnp.dot(x_r[...], wr_r[...], preferred_element_type=jnp.float32,
                       precision=lax.Precision.HIGHEST))
        if act == "relu":
            o_r[...] = jnp.maximum(h, 0.0)
        else:
            m = jnp.max(h, axis=1, keepdims=True)
            ex = jnp.exp(h - m)
            o_r[...] = ex / jnp.sum(ex, axis=1, keepdims=True)

    row_spec = pl.BlockSpec((blk, d), lambda i: (i, 0))
    deg_spec = pl.BlockSpec((blk, 16), lambda i: (i, 0))
    full_spec = pl.BlockSpec((d, d), lambda i: (0, 0))
    bias_spec = pl.BlockSpec((1, d), lambda i: (0, 0))
    return pl.pallas_call(
        body,
        grid=grid,
        in_specs=[row_spec, row_spec, deg_spec, deg_spec, row_spec,
                  full_spec, bias_spec, full_spec],
        out_specs=row_spec,
        out_shape=jax.ShapeDtypeStruct((n, d), jnp.float32),
    )


def kernel(x, edge_index, W1_l, b1, W1_r, W2_l, b2, W2_r):
    n, d = x.shape
    e = edge_index.shape[1]
    src2d = edge_index[0].reshape(e // CHUNK, CHUNK)
    dst2d = edge_index[1].reshape(e // CHUNK, CHUNK)
    zacc = jnp.zeros((n // NS, d), jnp.float32)
    zdeg = jnp.zeros((n // NS, 16), jnp.float32)
    ones = jnp.ones((CHUNK, 16), jnp.float32)

    agg1, deg = _make_aggregate(n, e, d, True)(x, src2d, dst2d, zacc, zdeg, ones)
    d0, d1 = deg[:n], deg[n:]
    h = _make_dense(n, d, "relu")(
        agg1[:n], agg1[n:], d0, d1, x, W1_l.T, b1.reshape(1, -1), W1_r.T)
    agg2 = _make_aggregate(n, e, d, False)(h, src2d, dst2d, zacc)
    out = _make_dense(n, d, "softmax")(
        agg2[0][:n], agg2[0][n:], d0, d1, h, W2_l.T, b2.reshape(1, -1), W2_r.T)
    return out
